# Initial kernel scaffold; baseline (speedup 1.0000x reference)
import jax
import jax.numpy as jnp
from jax.experimental import pallas as pl


def kernel(token_ids, embed_weight):
    raise NotImplementedError("write your pallas kernel here")


# speedup vs baseline: 1.8717x; 1.8717x over previous
import functools

import jax
import jax.numpy as jnp
from jax import lax
from jax.experimental import pallas as pl
from jax.experimental.pallas import tpu as pltpu
from jax.experimental.pallas import tpu_sc as plsc

CH = 128
K = 8


@functools.lru_cache(maxsize=None)
def _make_lookup(n_tok: int, vocab: int, emb: int):
    info = plsc.get_sparse_core_info()
    nc, ns = info.num_cores, info.num_subcores
    nw = nc * ns
    assert n_tok % (nw * CH * K) == 0
    nch = n_tok // (nw * CH)
    groups = nch // K

    mesh = plsc.VectorSubcoreMesh(core_axis_name="c", subcore_axis_name="s")

    @functools.partial(
        pl.kernel,
        mesh=mesh,
        out_type=jax.ShapeDtypeStruct((n_tok, emb), jnp.float32),
        compiler_params=pltpu.CompilerParams(use_tc_tiling_on_sc=False),
        scratch_types=(
            [pltpu.VMEM((nch, CH), jnp.int32),
             pltpu.VMEM((K, CH, emb), jnp.float32)]
            + [pltpu.SemaphoreType.DMA] * K
        ),
    )
    def lookup(ids_hbm, table_hbm, out_hbm, idx_v, rows_v, *sems):
        wid = lax.axis_index("s") * nc + lax.axis_index("c")
        cbase = wid * nch
        pltpu.sync_copy(ids_hbm.at[pl.ds(cbase, nch)], idx_v)

        def group(g, carry):
            first = g * K
            copies = []
            for b in range(K):
                copies.append(
                    pltpu.async_copy(
                        table_hbm.at[idx_v.at[first + b]],
                        rows_v.at[b],
                        sems[b],
                    )
                )
            for b in range(K):
                copies[b].wait()
                row0 = (cbase + first + b) * CH
                pltpu.sync_copy(rows_v.at[b], out_hbm.at[pl.ds(row0, CH)])
            return carry

        lax.fori_loop(0, groups, group, 0)

    return lookup


def kernel(token_ids, embed_weight):
    b, l = token_ids.shape
    vocab, emb = embed_weight.shape
    n_tok = b * l
    ids2d = token_ids.reshape(n_tok // CH, CH)
    out = _make_lookup(n_tok, vocab, emb)(ids2d, embed_weight)
    return out.reshape(b, l, emb)
